# baseline (speedup 1.0000x reference)
import functools

import jax
import jax.numpy as jnp
from jax import lax
from jax.experimental import pallas as pl
from jax.experimental.pallas import tpu as pltpu
from jax.experimental.pallas import tpu_sc as plsc

_NC = 2
_NS = 16
_NW = _NC * _NS

_TILE = 256
_RT = 512


def _router_body(x_ref, rw_ref, idx_ref, val_ref):
    xb = x_ref[...]
    rw = rw_ref[...]
    logits = lax.dot_general(xb, rw, (((1,), (1,)), ((), ())),
                             preferred_element_type=jnp.float32)
    z = logits - jnp.max(logits, axis=1, keepdims=True)
    ez = jnp.exp(z)
    probs = ez / jnp.sum(ez, axis=1, keepdims=True)
    n_exp = probs.shape[1]
    iota = lax.broadcasted_iota(jnp.int32, probs.shape, 1)
    m1 = jnp.max(probs, axis=1, keepdims=True)
    i1 = jnp.min(jnp.where(probs == m1, iota, n_exp), axis=1, keepdims=True)
    p2 = jnp.where(iota == i1, -jnp.inf, probs)
    m2 = jnp.max(p2, axis=1, keepdims=True)
    i2 = jnp.min(jnp.where(p2 == m2, iota, n_exp), axis=1, keepdims=True)
    idx_ref[...] = jnp.concatenate([i1, i2], axis=1)
    val_ref[...] = jnp.concatenate([m1, m2], axis=1)


def _router(xf, router_w):
    t, d = xf.shape
    e = router_w.shape[0]
    return pl.pallas_call(
        _router_body,
        grid=(t // _RT,),
        in_specs=[
            pl.BlockSpec((_RT, d), lambda i: (i, 0)),
            pl.BlockSpec((e, d), lambda i: (0, 0)),
        ],
        out_specs=[
            pl.BlockSpec((_RT, 2), lambda i: (i, 0)),
            pl.BlockSpec((_RT, 2), lambda i: (i, 0)),
        ],
        out_shape=[
            jax.ShapeDtypeStruct((t, 2), jnp.int32),
            jax.ShapeDtypeStruct((t, 2), jnp.float32),
        ],
    )(xf, router_w)


def _metadata(top_idx, n_experts, n_rows):
    ids = top_idx.reshape(-1)
    onehot = (ids[:, None] ==
              jnp.arange(n_experts, dtype=jnp.int32)[None, :]).astype(jnp.int32)
    counts = jnp.sum(onehot, axis=0)
    rank = jnp.cumsum(onehot, axis=0) - onehot
    rank = jnp.sum(rank * onehot, axis=1)
    padded = ((counts + _TILE - 1) // _TILE) * _TILE
    pad_end = jnp.cumsum(padded)
    pad_off = pad_end - padded
    rows = jnp.sum(onehot * pad_off[None, :], axis=1) + rank
    nb = n_rows // _TILE
    bounds = pad_end // _TILE
    raw = jnp.sum((jnp.arange(nb, dtype=jnp.int32)[:, None] >=
                   bounds[None, :]).astype(jnp.int32), axis=1)
    eidx = jnp.arange(n_experts, dtype=jnp.int32)
    last_e = jnp.max(jnp.where(counts > 0, eidx, -1))
    n_used = bounds[-1]
    block_used = (jnp.arange(nb, dtype=jnp.int32) < n_used).astype(jnp.int32)
    block_expert = jnp.where(raw >= n_experts, last_e, raw).astype(jnp.int32)
    return rows.astype(jnp.int32), block_expert, block_used


def _dispatch(xf, rows_e, rows_o, n_rows):
    t, d = xf.shape
    tpw = t // _NW
    mesh = plsc.VectorSubcoreMesh(core_axis_name="c", subcore_axis_name="s")

    @functools.partial(
        pl.kernel, mesh=mesh,
        out_type=jax.ShapeDtypeStruct((n_rows, d), jnp.float32),
        scratch_types=[
            pltpu.VMEM((tpw,), jnp.int32),
            pltpu.VMEM((tpw,), jnp.int32),
            pltpu.VMEM((tpw, d), jnp.float32),
            pltpu.SemaphoreType.DMA,
            pltpu.SemaphoreType.DMA,
        ],
    )
    def k(x_hbm, re_hbm, ro_hbm, out_hbm, idxe_v, idxo_v, buf, sem_e, sem_o):
        wid = lax.axis_index("s") * _NC + lax.axis_index("c")
        base = wid * tpw
        pltpu.sync_copy(re_hbm.at[pl.ds(base, tpw)], idxe_v)
        pltpu.sync_copy(ro_hbm.at[pl.ds(base, tpw)], idxo_v)
        pltpu.sync_copy(x_hbm.at[pl.ds(base, tpw)], buf)
        cp_e = pltpu.async_copy(buf, out_hbm.at[idxe_v], sem_e)
        cp_o = pltpu.async_copy(buf, out_hbm.at[idxo_v], sem_o)
        cp_e.wait()
        cp_o.wait()

    return k(xf, rows_e, rows_o)


def _mlp_body(be_ref, bu_ref, xs_ref, wg_ref, wi_ref, bg_ref, bi_ref,
              wo_ref, bo_ref, out_ref):
    del be_ref

    @pl.when(bu_ref[pl.program_id(0)] == 1)
    def _():
        xb = xs_ref[...]
        hg = lax.dot_general(xb, wg_ref[0], (((1,), (1,)), ((), ())),
                             preferred_element_type=jnp.float32) + bg_ref[0]
        hi = lax.dot_general(xb, wi_ref[0], (((1,), (1,)), ((), ())),
                             preferred_element_type=jnp.float32) + bi_ref[0]
        act = hg * lax.logistic(hg) * hi
        out_ref[...] = lax.dot_general(
            act, wo_ref[0], (((1,), (1,)), ((), ())),
            preferred_element_type=jnp.float32) + bo_ref[0]


def _grouped_mlp(xs, w_in, b_in, w_out, b_out, block_expert, block_used):
    n_rows, d = xs.shape
    e, f2, _ = w_in.shape
    f = f2 // 2
    nb = n_rows // _TILE
    grid_spec = pltpu.PrefetchScalarGridSpec(
        num_scalar_prefetch=2,
        grid=(nb,),
        in_specs=[
            pl.BlockSpec((_TILE, d), lambda i, be, bu: (bu[i] * i, 0)),
            pl.BlockSpec((1, f, d), lambda i, be, bu: (be[i], 0, 0)),
            pl.BlockSpec((1, f, d), lambda i, be, bu: (be[i], 1, 0)),
            pl.BlockSpec((1, 1, f), lambda i, be, bu: (2 * be[i], 0, 0)),
            pl.BlockSpec((1, 1, f), lambda i, be, bu: (2 * be[i] + 1, 0, 0)),
            pl.BlockSpec((1, d, f), lambda i, be, bu: (be[i], 0, 0)),
            pl.BlockSpec((1, 1, d), lambda i, be, bu: (be[i], 0, 0)),
        ],
        out_specs=pl.BlockSpec((_TILE, d), lambda i, be, bu: (i, 0)),
    )
    return pl.pallas_call(
        _mlp_body,
        grid_spec=grid_spec,
        out_shape=jax.ShapeDtypeStruct((n_rows, d), jnp.float32),
        compiler_params=pltpu.CompilerParams(
            dimension_semantics=("arbitrary",)),
    )(block_expert, block_used, xs, w_in, w_in, b_in.reshape(2 * e, 1, f),
      b_in.reshape(2 * e, 1, f), w_out, b_out.reshape(e, 1, d))


def _combine(out_rows, rows, gates, t):
    n_rows, d = out_rows.shape
    k_tk = rows.shape[0]
    tpw = t // _NW
    cht = 8
    nbuf = 3
    mesh = plsc.VectorSubcoreMesh(core_axis_name="c", subcore_axis_name="s")

    @functools.partial(
        pl.kernel, mesh=mesh,
        out_type=jax.ShapeDtypeStruct((t, d), jnp.float32),
        scratch_types=[
            pltpu.VMEM((nbuf, 2 * cht), jnp.int32),
            pltpu.VMEM((2 * cht, 16), jnp.float32),
            pltpu.VMEM((nbuf, 2 * cht, d), jnp.float32),
            pltpu.VMEM((2, cht, d), jnp.float32),
            pltpu.SemaphoreType.DMA,
            pltpu.SemaphoreType.DMA,
            pltpu.SemaphoreType.DMA,
            pltpu.SemaphoreType.DMA,
            pltpu.SemaphoreType.DMA,
        ],
    )
    def k(rows_hbm, inv_hbm, g_hbm, y_hbm, idx_v, g_v, r_v3, y_v2,
          sem_a, sem_b, sem_c, wsem_a, wsem_b):
        wid = lax.axis_index("s") * _NC + lax.axis_index("c")
        n_ch = tpw // cht
        sems = (sem_a, sem_b, sem_c)
        wsems = (wsem_a, wsem_b)

        cps = {}

        def start_gather(c):
            b = c % nbuf
            pltpu.sync_copy(
                inv_hbm.at[pl.ds(2 * (wid * tpw + c * cht), 2 * cht)],
                idx_v.at[b])
            cps[c] = pltpu.async_copy(
                rows_hbm.at[idx_v.at[b]], r_v3.at[b], sems[b])

        start_gather(0)
        start_gather(1)
        wcps = {}
        for c in range(n_ch):
            tbase = wid * tpw + c * cht
            if c + 2 < n_ch:
                start_gather(c + 2)
            pltpu.sync_copy(g_hbm.at[pl.ds(2 * tbase, 2 * cht)], g_v)
            cps[c].wait()
            if c >= 2:
                wcps[c - 2].wait()
            r_v = r_v3.at[c % nbuf]
            y_v = y_v2.at[c % 2]

            def body(tt, carry):
                g0 = g_v[2 * tt]
                g1 = g_v[2 * tt + 1]
                for dc in range(d // 16):
                    sl = pl.ds(dc * 16, 16)
                    y_v[tt, sl] = g0 * r_v[2 * tt, sl] + g1 * r_v[2 * tt + 1, sl]
                return carry

            lax.fori_loop(0, cht, body, 0)
            wcps[c] = pltpu.async_copy(
                y_v, y_hbm.at[pl.ds(tbase, cht)], wsems[c % 2])
        wcps[n_ch - 2].wait()
        wcps[n_ch - 1].wait()

    return k(out_rows, rows, jnp.broadcast_to(gates[:, None], (k_tk, 16)))


def kernel(x, router_w, w_in, b_in, w_out, b_out):
    bq, sq, d = x.shape
    t = bq * sq
    e = router_w.shape[0]
    k = 2
    xf = x.reshape(t, d)

    top_idx, top_val = _router(xf, router_w)

    n_rows = ((t * k + e * (_TILE - 1)) // _TILE) * _TILE
    rows, block_expert, block_used = _metadata(top_idx, e, n_rows)
    rows2 = rows.reshape(t, k)
    rows_e = rows2[:, 0]
    rows_o = rows2[:, 1]

    xs = _dispatch(xf, rows_e, rows_o, n_rows)
    out_rows = _grouped_mlp(xs, w_in, b_in, w_out, b_out, block_expert,
                            block_used)
    y = _combine(out_rows, rows, top_val.reshape(-1), t)
    return y.reshape(bq, sq, d)


# speedup vs baseline: 1.0158x; 1.0158x over previous
import functools

import jax
import jax.numpy as jnp
from jax import lax
from jax.experimental import pallas as pl
from jax.experimental.pallas import tpu as pltpu
from jax.experimental.pallas import tpu_sc as plsc

_NC = 2
_NS = 16
_NW = _NC * _NS

_TILE = 256
_RT = 512


def _router_body(x_ref, rw_ref, idx_ref, val_ref):
    xb = x_ref[...]
    rw = rw_ref[...]
    logits = lax.dot_general(xb, rw, (((1,), (1,)), ((), ())),
                             preferred_element_type=jnp.float32)
    z = logits - jnp.max(logits, axis=1, keepdims=True)
    ez = jnp.exp(z)
    probs = ez / jnp.sum(ez, axis=1, keepdims=True)
    n_exp = probs.shape[1]
    iota = lax.broadcasted_iota(jnp.int32, probs.shape, 1)
    m1 = jnp.max(probs, axis=1, keepdims=True)
    i1 = jnp.min(jnp.where(probs == m1, iota, n_exp), axis=1, keepdims=True)
    p2 = jnp.where(iota == i1, -jnp.inf, probs)
    m2 = jnp.max(p2, axis=1, keepdims=True)
    i2 = jnp.min(jnp.where(p2 == m2, iota, n_exp), axis=1, keepdims=True)
    idx_ref[...] = jnp.concatenate([i1, i2], axis=1)
    val_ref[...] = jnp.concatenate([m1, m2], axis=1)


def _router(xf, router_w):
    t, d = xf.shape
    e = router_w.shape[0]
    return pl.pallas_call(
        _router_body,
        grid=(t // _RT,),
        in_specs=[
            pl.BlockSpec((_RT, d), lambda i: (i, 0)),
            pl.BlockSpec((e, d), lambda i: (0, 0)),
        ],
        out_specs=[
            pl.BlockSpec((_RT, 2), lambda i: (i, 0)),
            pl.BlockSpec((_RT, 2), lambda i: (i, 0)),
        ],
        out_shape=[
            jax.ShapeDtypeStruct((t, 2), jnp.int32),
            jax.ShapeDtypeStruct((t, 2), jnp.float32),
        ],
    )(xf, router_w)


def _metadata(top_idx, n_experts, n_rows):
    ids = top_idx.reshape(-1)
    onehot = (ids[:, None] ==
              jnp.arange(n_experts, dtype=jnp.int32)[None, :]).astype(jnp.int32)
    counts = jnp.sum(onehot, axis=0)
    rank = jnp.cumsum(onehot, axis=0) - onehot
    rank = jnp.sum(rank * onehot, axis=1)
    padded = ((counts + _TILE - 1) // _TILE) * _TILE
    pad_end = jnp.cumsum(padded)
    pad_off = pad_end - padded
    rows = jnp.sum(onehot * pad_off[None, :], axis=1) + rank
    nb = n_rows // _TILE
    bounds = pad_end // _TILE
    raw = jnp.sum((jnp.arange(nb, dtype=jnp.int32)[:, None] >=
                   bounds[None, :]).astype(jnp.int32), axis=1)
    eidx = jnp.arange(n_experts, dtype=jnp.int32)
    last_e = jnp.max(jnp.where(counts > 0, eidx, -1))
    n_used = bounds[-1]
    block_used = (jnp.arange(nb, dtype=jnp.int32) < n_used).astype(jnp.int32)
    block_expert = jnp.where(raw >= n_experts, last_e, raw).astype(jnp.int32)
    return rows.astype(jnp.int32), block_expert, block_used


def _dispatch(xf, rows_e, rows_o, n_rows):
    t, d = xf.shape
    tpw = t // _NW
    mesh = plsc.VectorSubcoreMesh(core_axis_name="c", subcore_axis_name="s")

    @functools.partial(
        pl.kernel, mesh=mesh,
        out_type=jax.ShapeDtypeStruct((n_rows, d), jnp.float32),
        scratch_types=[
            pltpu.VMEM((tpw,), jnp.int32),
            pltpu.VMEM((tpw,), jnp.int32),
            pltpu.VMEM((tpw, d), jnp.float32),
            pltpu.SemaphoreType.DMA,
            pltpu.SemaphoreType.DMA,
        ],
    )
    def k(x_hbm, re_hbm, ro_hbm, out_hbm, idxe_v, idxo_v, buf, sem_e, sem_o):
        wid = lax.axis_index("s") * _NC + lax.axis_index("c")
        base = wid * tpw
        pltpu.sync_copy(re_hbm.at[pl.ds(base, tpw)], idxe_v)
        pltpu.sync_copy(ro_hbm.at[pl.ds(base, tpw)], idxo_v)
        pltpu.sync_copy(x_hbm.at[pl.ds(base, tpw)], buf)
        cp_e = pltpu.async_copy(buf, out_hbm.at[idxe_v], sem_e)
        cp_o = pltpu.async_copy(buf, out_hbm.at[idxo_v], sem_o)
        cp_e.wait()
        cp_o.wait()

    return k(xf, rows_e, rows_o)


def _mlp_body(be_ref, bu_ref, xs_ref, wg_ref, wi_ref, bg_ref, bi_ref,
              wo_ref, bo_ref, out_ref):
    del be_ref

    @pl.when(bu_ref[pl.program_id(0)] == 1)
    def _():
        xb = xs_ref[...]
        hg = lax.dot_general(xb, wg_ref[0], (((1,), (1,)), ((), ())),
                             preferred_element_type=jnp.float32) + bg_ref[0]
        hi = lax.dot_general(xb, wi_ref[0], (((1,), (1,)), ((), ())),
                             preferred_element_type=jnp.float32) + bi_ref[0]
        act = hg * lax.logistic(hg) * hi
        out_ref[...] = lax.dot_general(
            act, wo_ref[0], (((1,), (1,)), ((), ())),
            preferred_element_type=jnp.float32) + bo_ref[0]


def _grouped_mlp(xs, w_in, b_in, w_out, b_out, block_expert, block_used):
    n_rows, d = xs.shape
    e, f2, _ = w_in.shape
    f = f2 // 2
    nb = n_rows // _TILE
    grid_spec = pltpu.PrefetchScalarGridSpec(
        num_scalar_prefetch=2,
        grid=(nb,),
        in_specs=[
            pl.BlockSpec((_TILE, d), lambda i, be, bu: (bu[i] * i, 0)),
            pl.BlockSpec((1, f, d), lambda i, be, bu: (be[i], 0, 0)),
            pl.BlockSpec((1, f, d), lambda i, be, bu: (be[i], 1, 0)),
            pl.BlockSpec((1, 1, f), lambda i, be, bu: (2 * be[i], 0, 0)),
            pl.BlockSpec((1, 1, f), lambda i, be, bu: (2 * be[i] + 1, 0, 0)),
            pl.BlockSpec((1, d, f), lambda i, be, bu: (be[i], 0, 0)),
            pl.BlockSpec((1, 1, d), lambda i, be, bu: (be[i], 0, 0)),
        ],
        out_specs=pl.BlockSpec((_TILE, d), lambda i, be, bu: (i, 0)),
    )
    return pl.pallas_call(
        _mlp_body,
        grid_spec=grid_spec,
        out_shape=jax.ShapeDtypeStruct((n_rows, d), jnp.float32),
        compiler_params=pltpu.CompilerParams(
            dimension_semantics=("arbitrary",)),
    )(block_expert, block_used, xs, w_in, w_in, b_in.reshape(2 * e, 1, f),
      b_in.reshape(2 * e, 1, f), w_out, b_out.reshape(e, 1, d))


def _combine(out_rows, rows, gates, t):
    n_rows, d = out_rows.shape
    k_tk = rows.shape[0]
    tpw = t // _NW
    cht = 16
    nbuf = 2
    mesh = plsc.VectorSubcoreMesh(core_axis_name="c", subcore_axis_name="s")

    @functools.partial(
        pl.kernel, mesh=mesh,
        out_type=jax.ShapeDtypeStruct((t, d), jnp.float32),
        scratch_types=[
            pltpu.VMEM((nbuf, 2 * cht), jnp.int32),
            pltpu.VMEM((2 * cht, 16), jnp.float32),
            pltpu.VMEM((nbuf, 2 * cht, d), jnp.float32),
            pltpu.VMEM((2, cht, d), jnp.float32),
            pltpu.SemaphoreType.DMA,
            pltpu.SemaphoreType.DMA,
            pltpu.SemaphoreType.DMA,
            pltpu.SemaphoreType.DMA,
        ],
    )
    def k(rows_hbm, inv_hbm, g_hbm, y_hbm, idx_v, g_v, r_v3, y_v2,
          sem_a, sem_b, wsem_a, wsem_b):
        wid = lax.axis_index("s") * _NC + lax.axis_index("c")
        n_ch = tpw // cht
        sems = (sem_a, sem_b)
        wsems = (wsem_a, wsem_b)

        cps = {}

        def start_gather(c):
            b = c % nbuf
            pltpu.sync_copy(
                inv_hbm.at[pl.ds(2 * (wid * tpw + c * cht), 2 * cht)],
                idx_v.at[b])
            cps[c] = pltpu.async_copy(
                rows_hbm.at[idx_v.at[b]], r_v3.at[b], sems[b])

        start_gather(0)
        wcps = {}
        for c in range(n_ch):
            tbase = wid * tpw + c * cht
            if c + 1 < n_ch:
                start_gather(c + 1)
            pltpu.sync_copy(g_hbm.at[pl.ds(2 * tbase, 2 * cht)], g_v)
            cps[c].wait()
            if c >= 2:
                wcps[c - 2].wait()
            r_v = r_v3.at[c % nbuf]
            y_v = y_v2.at[c % 2]

            def body(tt, carry):
                g0 = g_v[2 * tt]
                g1 = g_v[2 * tt + 1]
                for dc in range(d // 16):
                    sl = pl.ds(dc * 16, 16)
                    y_v[tt, sl] = g0 * r_v[2 * tt, sl] + g1 * r_v[2 * tt + 1, sl]
                return carry

            lax.fori_loop(0, cht, body, 0)
            wcps[c] = pltpu.async_copy(
                y_v, y_hbm.at[pl.ds(tbase, cht)], wsems[c % 2])
        wcps[n_ch - 2].wait()
        wcps[n_ch - 1].wait()

    return k(out_rows, rows, jnp.broadcast_to(gates[:, None], (k_tk, 16)))


def kernel(x, router_w, w_in, b_in, w_out, b_out):
    bq, sq, d = x.shape
    t = bq * sq
    e = router_w.shape[0]
    k = 2
    xf = x.reshape(t, d)

    top_idx, top_val = _router(xf, router_w)

    n_rows = ((t * k + e * (_TILE - 1)) // _TILE) * _TILE
    rows, block_expert, block_used = _metadata(top_idx, e, n_rows)
    rows2 = rows.reshape(t, k)
    rows_e = rows2[:, 0]
    rows_o = rows2[:, 1]

    xs = _dispatch(xf, rows_e, rows_o, n_rows)
    out_rows = _grouped_mlp(xs, w_in, b_in, w_out, b_out, block_expert,
                            block_used)
    y = _combine(out_rows, rows, top_val.reshape(-1), t)
    return y.reshape(bq, sq, d)
